# baseline (speedup 1.0000x reference)
import functools

import jax
import jax.numpy as jnp
from jax import lax
from jax.experimental import pallas as pl
from jax.experimental.pallas import tpu as pltpu
from jax.experimental.pallas import tpu_sc as plsc

CONV = 64
NC = 2
NS = 16
L = 16
EPG = 768
SBG = 2
NPAD = 49 * 2048
TB = 2048

_SC_PARAMS = None


def _sc_mesh():
    return plsc.VectorSubcoreMesh(core_axis_name="c", subcore_axis_name="s",
                                  num_cores=NC, num_subcores=NS)


def _sc_compiler_params():
    return pltpu.CompilerParams(use_tc_tiling_on_sc=False)


def _zero_acc_slice(acc, buf, s, SL):
    @pl.loop(0, EPG)
    def _zero(i):
        buf[i, :] = jnp.zeros((L,), jnp.float32)

    full, rem = SL // EPG, SL % EPG
    for k in range(full):
        pltpu.sync_copy(buf, acc.at[pl.ds(s * SL + k * EPG, EPG)])
    if rem:
        pltpu.sync_copy(buf.at[pl.ds(0, rem)],
                        acc.at[pl.ds(s * SL + full * EPG, rem)])


def _sc_degree(dst2):
    RTOT = dst2.shape[0]
    R_sc = RTOT // NC
    RT = R_sc // NS
    SL = NPAD // NS

    @functools.partial(
        pl.kernel,
        out_type=jax.ShapeDtypeStruct((NC, NPAD, L), jnp.float32),
        mesh=_sc_mesh(),
        scratch_types=[
            pltpu.VMEM_SHARED((NPAD, L), jnp.float32),
            pltpu.VMEM((EPG, L), jnp.float32),
            pltpu.VMEM((2, EPG), jnp.int32),
            pltpu.SemaphoreType.DMA,
            pltpu.SemaphoreType.DMA,
        ],
        compiler_params=_sc_compiler_params(),
    )
    def deg_kernel(dst_hbm, out_hbm, acc, ones, dstv, semA, semB):
        c = lax.axis_index("c")
        s = lax.axis_index("s")

        _zero_acc_slice(acc, ones, s, SL)

        @pl.loop(0, EPG)
        def _fill(i):
            ones[i, :] = jnp.full((L,), 1.0, jnp.float32)

        plsc.subcore_barrier()

        base = c * R_sc + s * RT
        sem = (semA, semB)

        @pl.loop(0, RT // 2)
        def _edges(m):
            r0 = base + m * 2
            pltpu.sync_copy(dst_hbm.at[pl.ds(r0, 2)], dstv)
            for b in range(2):
                pltpu.async_copy(ones, acc.at[dstv.at[b]], sem[b], add=True)
            for b in range(2):
                pltpu.make_async_copy(ones, acc.at[dstv.at[b]],
                                      sem[b]).wait()

        plsc.subcore_barrier()
        pltpu.sync_copy(acc.at[pl.ds(s * SL, SL)],
                        out_hbm.at[c].at[pl.ds(s * SL, SL)])

    return deg_kernel(dst2)


def _sc_edge_agg(table, src2, dst2):
    RTOT = src2.shape[0]
    RT = RTOT // NS
    SB = RT // SBG
    SL = NPAD // NS
    TSPAN = NPAD * 8 - 7

    @functools.partial(
        pl.kernel,
        out_type=jax.ShapeDtypeStruct((NPAD, 128), jnp.float32),
        mesh=_sc_mesh(),
        scratch_types=[
            pltpu.VMEM_SHARED((NPAD, L), jnp.float32),
            pltpu.VMEM((EPG, L), jnp.float32),
            pltpu.VMEM((EPG, L), jnp.float32),
            pltpu.VMEM((SBG, EPG), jnp.int32),
            pltpu.VMEM((SBG, EPG), jnp.int32),
            pltpu.VMEM((SBG, EPG), jnp.int32),
            pltpu.SemaphoreType.DMA,
            pltpu.SemaphoreType.DMA,
            pltpu.SemaphoreType.DMA,
            pltpu.SemaphoreType.DMA,
        ],
        compiler_params=_sc_compiler_params(),
    )
    def agg_kernel(table_hbm, src_hbm, dst_hbm, out_hbm,
                   acc, rows0, rows1, srcv, dstv0, dstv1,
                   gsem0, gsem1, ssem0, ssem1):
        c = lax.axis_index("c")
        s = lax.axis_index("s")
        rows = (rows0, rows1)
        dstv = (dstv0, dstv1)
        gsem = (gsem0, gsem1)
        ssem = (ssem0, ssem1)
        base = s * RT

        for p in range(2):
            q = c * 2 + p

            _zero_acc_slice(acc, rows0, s, SL)
            plsc.subcore_barrier()

            tab = table_hbm.at[pl.ds(q, TSPAN)]

            def fetch(m, ip):
                r0 = base + m * SBG
                pltpu.sync_copy(src_hbm.at[pl.ds(r0, SBG)], srcv)
                pltpu.sync_copy(dst_hbm.at[pl.ds(r0, SBG)], dstv[ip])
                for j in range(SBG):
                    @pl.loop(0, EPG // L, unroll=8)
                    def _adj(i):
                        srcv[j, pl.ds(i * L, L)] = (
                            srcv[j, pl.ds(i * L, L)] * 8)

            def g_fire(b, j):
                pltpu.async_copy(tab.at[srcv.at[j]], rows[b], gsem[b])

            def g_wait(b):
                pltpu.make_async_copy(tab.at[srcv.at[0]], rows[b],
                                      gsem[b]).wait()

            def s_fire(b, ip, j):
                pltpu.async_copy(rows[b], acc.at[dstv[ip].at[j]],
                                 ssem[b], add=True)

            def s_wait(b):
                pltpu.make_async_copy(rows[b], acc.at[dstv[0].at[0]],
                                      ssem[b]).wait()

            def sb_body(m, ip):
                g_wait(1)
                s_fire(1, 1 - ip, 1)
                fetch(m, ip)
                s_wait(0)
                g_fire(0, 0)
                s_wait(1)
                g_fire(1, 1)
                g_wait(0)
                s_fire(0, ip, 0)

            fetch(0, 0)
            g_fire(0, 0)
            g_fire(1, 1)
            g_wait(0)
            s_fire(0, 0, 0)

            @pl.loop(0, (SB - 1) // 2)
            def _edges(t):
                sb_body(2 * t + 1, 1)
                sb_body(2 * t + 2, 0)

            if (SB - 1) % 2:
                sb_body(SB - 1, (SB - 1) % 2)

            g_wait(1)
            s_fire(1, (SB - 1) % 2, 1)
            s_wait(0)
            s_wait(1)

            plsc.subcore_barrier()
            pltpu.sync_copy(acc.at[pl.ds(s * SL, SL)],
                            out_hbm.at[pl.ds(s * SL, SL), pl.ds(q * L, L)])
            plsc.subcore_barrier()

    return agg_kernel(table, src2, dst2)


def _layernorm(h, g, b):
    m = jnp.mean(h, axis=-1, keepdims=True)
    v = jnp.mean((h - m) * (h - m), axis=-1, keepdims=True)
    return (h - m) * lax.rsqrt(v + 1e-5) * g + b


def _dot(a, b):
    return jnp.dot(a, b, preferred_element_type=jnp.float32)


def _tc_stage1(coords_p, x_p, dacc, W_se0, b_se0, g_se0, be_se0,
               W_se1, b_se1, W_d0, b_d0, g_d0, be_d0, W_d1, b_d1, W1x, W1e):
    grid = NPAD // TB

    def body(coords_ref, x_ref, dacc_ref, Wse0_ref, bse0_ref, gse0_ref,
             bese0_ref, Wse1_ref, bse1_ref, Wd0_ref, bd0_ref, gd0_ref,
             bed0_ref, Wd1_ref, bd1_ref, W1x_ref, W1e_ref,
             hst_ref, dinv_ref):
        h = jax.nn.relu(_dot(coords_ref[...], Wse0_ref[...]) + bse0_ref[...])
        h = _layernorm(h, gse0_ref[...], bese0_ref[...])
        h = jax.nn.relu(_dot(h, Wse1_ref[...]) + bse1_ref[...])
        d = jax.nn.relu(_dot(h, Wd0_ref[...]) + bd0_ref[...])
        d = _layernorm(d, gd0_ref[...], bed0_ref[...])
        emb = jax.nn.relu(_dot(d, Wd1_ref[...]) + bd1_ref[...])
        hw1 = _dot(x_ref[...], W1x_ref[...]) + _dot(emb, W1e_ref[...])
        deg = dacc_ref[0, :, :1] + dacc_ref[1, :, :1] + 1.0
        dinv = lax.rsqrt(deg)
        hst_ref[:, :CONV] = hw1 * dinv
        hst_ref[:, CONV:] = hw1 * (dinv * dinv)
        dinv_ref[...] = dinv

    fullspec = lambda shape: pl.BlockSpec(shape, lambda i: (0,) * len(shape))
    return pl.pallas_call(
        body,
        grid=(grid,),
        in_specs=[
            pl.BlockSpec((TB, 2), lambda i: (i, 0)),
            pl.BlockSpec((TB, 6), lambda i: (i, 0)),
            pl.BlockSpec((NC, TB, L), lambda i: (0, i, 0)),
            fullspec((2, 128)), fullspec((1, 128)), fullspec((1, 128)),
            fullspec((1, 128)), fullspec((128, 128)), fullspec((1, 128)),
            fullspec((128, 64)), fullspec((1, 64)), fullspec((1, 64)),
            fullspec((1, 64)), fullspec((64, 16)), fullspec((1, 16)),
            fullspec((6, 64)), fullspec((16, 64)),
        ],
        out_specs=[
            pl.BlockSpec((TB, 128), lambda i: (i, 0)),
            pl.BlockSpec((TB, 1), lambda i: (i, 0)),
        ],
        out_shape=[
            jax.ShapeDtypeStruct((NPAD, 128), jnp.float32),
            jax.ShapeDtypeStruct((NPAD, 1), jnp.float32),
        ],
    )(coords_p, x_p, dacc, W_se0, b_se0, g_se0, be_se0, W_se1, b_se1,
      W_d0, b_d0, g_d0, be_d0, W_d1, b_d1, W1x, W1e)


def _tc_stage2(acc1, hst1, dinv, b1, W2):
    grid = NPAD // TB

    def body(acc_ref, hst_ref, dinv_ref, b1_ref, W2_ref, hst2_ref):
        dinv = dinv_ref[...]
        z1 = jax.nn.relu(acc_ref[:, :CONV] * dinv + hst_ref[:, CONV:]
                         + b1_ref[...])
        hw2 = _dot(z1, W2_ref[...])
        hst2_ref[:, :CONV] = hw2 * dinv
        hst2_ref[:, CONV:] = hw2 * (dinv * dinv)

    fullspec = lambda shape: pl.BlockSpec(shape, lambda i: (0,) * len(shape))
    return pl.pallas_call(
        body,
        grid=(grid,),
        in_specs=[
            pl.BlockSpec((TB, 128), lambda i: (i, 0)),
            pl.BlockSpec((TB, 128), lambda i: (i, 0)),
            pl.BlockSpec((TB, 1), lambda i: (i, 0)),
            fullspec((1, 64)), fullspec((64, 64)),
        ],
        out_specs=pl.BlockSpec((TB, 128), lambda i: (i, 0)),
        out_shape=jax.ShapeDtypeStruct((NPAD, 128), jnp.float32),
    )(acc1, hst1, dinv, b1, W2)


def _tc_stage3(acc2, hst2, dinv, b2, Wfc, bfc, n):
    grid = NPAD // TB

    def body(acc_ref, hst_ref, dinv_ref, b2_ref, Wfc_ref, bfc_ref, out_ref):
        dinv = dinv_ref[...]
        z2 = jax.nn.relu(acc_ref[:, :CONV] * dinv + hst_ref[:, CONV:]
                         + b2_ref[...])
        out_ref[...] = _dot(z2, Wfc_ref[...]) + bfc_ref[...]

    fullspec = lambda shape: pl.BlockSpec(shape, lambda i: (0,) * len(shape))
    return pl.pallas_call(
        body,
        grid=(grid,),
        in_specs=[
            pl.BlockSpec((TB, 128), lambda i: (i, 0)),
            pl.BlockSpec((TB, 128), lambda i: (i, 0)),
            pl.BlockSpec((TB, 1), lambda i: (i, 0)),
            fullspec((1, 64)), fullspec((64, 1)), fullspec((1, 1)),
        ],
        out_specs=pl.BlockSpec((TB, 1), lambda i: (i, 0)),
        out_shape=jax.ShapeDtypeStruct((n, 1), jnp.float32),
    )(acc2, hst2, dinv, b2, Wfc, bfc)


def kernel(x, coords, edge_index, W_se0, b_se0, g_se0, be_se0, W_se1, b_se1,
           W_d0, b_d0, g_d0, be_d0, W_d1, b_d1, W1, b1, W2, b2, Wfc, bfc):
    n = x.shape[0]
    e = edge_index.shape[1]
    assert n < NPAD

    coords_p = jnp.zeros((NPAD, 2), jnp.float32).at[:n].set(coords)
    x_p = jnp.zeros((NPAD, 6), jnp.float32).at[:n].set(x)

    epad = ((e + 2 * NS * EPG - 1) // (2 * NS * EPG)) * (2 * NS * EPG)
    src = jnp.pad(edge_index[0], (0, epad - e),
                  constant_values=n).reshape(epad // EPG, EPG)
    dst = jnp.pad(edge_index[1], (0, epad - e),
                  constant_values=n).reshape(epad // EPG, EPG)

    row = lambda v: v.reshape(1, -1)

    dacc = _sc_degree(dst)
    hst1, dinv = _tc_stage1(
        coords_p, x_p, dacc,
        W_se0, row(b_se0), row(g_se0), row(be_se0),
        W_se1, row(b_se1), W_d0, row(b_d0), row(g_d0), row(be_d0),
        W_d1, row(b_d1), W1[:6], W1[6:])

    acc1 = _sc_edge_agg(hst1.reshape(NPAD * 8, L), src, dst)
    hst2 = _tc_stage2(acc1, hst1, dinv, row(b1), W2)

    acc2 = _sc_edge_agg(hst2.reshape(NPAD * 8, L), src, dst)
    return _tc_stage3(acc2, hst2, dinv, row(b2), Wfc, row(bfc), n)


# speedup vs baseline: 1.2425x; 1.2425x over previous
import functools

import jax
import jax.numpy as jnp
from jax import lax
from jax.experimental import pallas as pl
from jax.experimental.pallas import tpu as pltpu
from jax.experimental.pallas import tpu_sc as plsc

CONV = 64
NC = 2
NS = 16
L = 16
EPG = 512
SBG = 4
NPAD = 49 * 2048
TB = 2048

_SC_PARAMS = None


def _sc_mesh():
    return plsc.VectorSubcoreMesh(core_axis_name="c", subcore_axis_name="s",
                                  num_cores=NC, num_subcores=NS)


def _sc_compiler_params():
    return pltpu.CompilerParams(use_tc_tiling_on_sc=False)


def _zero_acc_slice(acc, buf, s, SL):
    @pl.loop(0, EPG)
    def _zero(i):
        buf[i, :] = jnp.zeros((L,), jnp.float32)

    full, rem = SL // EPG, SL % EPG
    for k in range(full):
        pltpu.sync_copy(buf, acc.at[pl.ds(s * SL + k * EPG, EPG)])
    if rem:
        pltpu.sync_copy(buf.at[pl.ds(0, rem)],
                        acc.at[pl.ds(s * SL + full * EPG, rem)])


def _sc_degree(dst2):
    RTOT = dst2.shape[0]
    R_sc = RTOT // NC
    RT = R_sc // NS
    SL = NPAD // NS

    @functools.partial(
        pl.kernel,
        out_type=jax.ShapeDtypeStruct((NC, NPAD, L), jnp.float32),
        mesh=_sc_mesh(),
        scratch_types=[
            pltpu.VMEM_SHARED((NPAD, L), jnp.float32),
            pltpu.VMEM((EPG, L), jnp.float32),
            pltpu.VMEM((2, EPG), jnp.int32),
            pltpu.SemaphoreType.DMA,
            pltpu.SemaphoreType.DMA,
        ],
        compiler_params=_sc_compiler_params(),
    )
    def deg_kernel(dst_hbm, out_hbm, acc, ones, dstv, semA, semB):
        c = lax.axis_index("c")
        s = lax.axis_index("s")

        _zero_acc_slice(acc, ones, s, SL)

        @pl.loop(0, EPG)
        def _fill(i):
            ones[i, :] = jnp.full((L,), 1.0, jnp.float32)

        plsc.subcore_barrier()

        base = c * R_sc + s * RT
        sem = (semA, semB)

        @pl.loop(0, RT // 2)
        def _edges(m):
            r0 = base + m * 2
            pltpu.sync_copy(dst_hbm.at[pl.ds(r0, 2)], dstv)
            for b in range(2):
                pltpu.async_copy(ones, acc.at[dstv.at[b]], sem[b], add=True)
            for b in range(2):
                pltpu.make_async_copy(ones, acc.at[dstv.at[b]],
                                      sem[b]).wait()

        plsc.subcore_barrier()
        pltpu.sync_copy(acc.at[pl.ds(s * SL, SL)],
                        out_hbm.at[c].at[pl.ds(s * SL, SL)])

    return deg_kernel(dst2)


def _sc_edge_agg(table, src2, dst2):
    RTOT = src2.shape[0]
    RT = RTOT // NS
    SB = RT // SBG
    SL = NPAD // NS
    TSPAN = NPAD * 8 - 7

    @functools.partial(
        pl.kernel,
        out_type=jax.ShapeDtypeStruct((NPAD, 128), jnp.float32),
        mesh=_sc_mesh(),
        scratch_types=[
            pltpu.VMEM_SHARED((NPAD, L), jnp.float32),
            pltpu.VMEM((EPG, L), jnp.float32),
            pltpu.VMEM((EPG, L), jnp.float32),
            pltpu.VMEM((SBG, EPG), jnp.int32),
            pltpu.VMEM((SBG, EPG), jnp.int32),
            pltpu.VMEM((SBG, EPG), jnp.int32),
            pltpu.SemaphoreType.DMA,
            pltpu.SemaphoreType.DMA,
            pltpu.SemaphoreType.DMA,
            pltpu.SemaphoreType.DMA,
        ],
        compiler_params=_sc_compiler_params(),
    )
    def agg_kernel(table_hbm, src_hbm, dst_hbm, out_hbm,
                   acc, rows0, rows1, srcv, dstv0, dstv1,
                   gsem0, gsem1, ssem0, ssem1):
        c = lax.axis_index("c")
        s = lax.axis_index("s")
        rows = (rows0, rows1)
        dstv = (dstv0, dstv1)
        gsem = (gsem0, gsem1)
        ssem = (ssem0, ssem1)
        base = s * RT

        for p in range(2):
            q = c * 2 + p

            _zero_acc_slice(acc, rows0, s, SL)
            plsc.subcore_barrier()

            tab = table_hbm.at[pl.ds(q, TSPAN)]

            def fetch(m, ip):
                r0 = base + m * SBG
                pltpu.sync_copy(src_hbm.at[pl.ds(r0, SBG)], srcv)
                pltpu.sync_copy(dst_hbm.at[pl.ds(r0, SBG)], dstv[ip])
                for j in range(SBG):
                    @pl.loop(0, EPG // L, unroll=8)
                    def _adj(i):
                        srcv[j, pl.ds(i * L, L)] = (
                            srcv[j, pl.ds(i * L, L)] * 8)

            def g_fire(b, j):
                pltpu.async_copy(tab.at[srcv.at[j]], rows[b], gsem[b])

            def g_wait(b):
                pltpu.make_async_copy(tab.at[srcv.at[0]], rows[b],
                                      gsem[b]).wait()

            def s_fire(b, ip, j):
                pltpu.async_copy(rows[b], acc.at[dstv[ip].at[j]],
                                 ssem[b], add=True)

            def s_wait(b):
                pltpu.make_async_copy(rows[b], acc.at[dstv[0].at[0]],
                                      ssem[b]).wait()

            def sb_body(m, ip):
                g_wait(1)
                s_fire(1, 1 - ip, SBG - 1)
                fetch(m, ip)
                s_wait(0)
                g_fire(0, 0)
                s_wait(1)
                g_fire(1, 1)
                g_wait(0)
                s_fire(0, ip, 0)
                s_wait(0)
                g_fire(0, 2)
                g_wait(1)
                s_fire(1, ip, 1)
                s_wait(1)
                g_fire(1, 3)
                g_wait(0)
                s_fire(0, ip, 2)

            fetch(0, 0)
            g_fire(0, 0)
            g_fire(1, 1)
            g_wait(0)
            s_fire(0, 0, 0)
            s_wait(0)
            g_fire(0, 2)
            g_wait(1)
            s_fire(1, 0, 1)
            s_wait(1)
            g_fire(1, 3)
            g_wait(0)
            s_fire(0, 0, 2)

            @pl.loop(0, (SB - 1) // 2)
            def _edges(t):
                sb_body(2 * t + 1, 1)
                sb_body(2 * t + 2, 0)

            if (SB - 1) % 2:
                sb_body(SB - 1, (SB - 1) % 2)

            g_wait(1)
            s_fire(1, (SB - 1) % 2, SBG - 1)
            s_wait(0)
            s_wait(1)

            plsc.subcore_barrier()
            pltpu.sync_copy(acc.at[pl.ds(s * SL, SL)],
                            out_hbm.at[pl.ds(s * SL, SL), pl.ds(q * L, L)])
            plsc.subcore_barrier()

    return agg_kernel(table, src2, dst2)


def _layernorm(h, g, b):
    m = jnp.mean(h, axis=-1, keepdims=True)
    v = jnp.mean((h - m) * (h - m), axis=-1, keepdims=True)
    return (h - m) * lax.rsqrt(v + 1e-5) * g + b


def _dot(a, b):
    return jnp.dot(a, b, preferred_element_type=jnp.float32)


def _tc_stage1(coords_p, x_p, dacc, W_se0, b_se0, g_se0, be_se0,
               W_se1, b_se1, W_d0, b_d0, g_d0, be_d0, W_d1, b_d1, W1x, W1e):
    grid = NPAD // TB

    def body(coords_ref, x_ref, dacc_ref, Wse0_ref, bse0_ref, gse0_ref,
             bese0_ref, Wse1_ref, bse1_ref, Wd0_ref, bd0_ref, gd0_ref,
             bed0_ref, Wd1_ref, bd1_ref, W1x_ref, W1e_ref,
             hst_ref, dinv_ref):
        h = jax.nn.relu(_dot(coords_ref[...], Wse0_ref[...]) + bse0_ref[...])
        h = _layernorm(h, gse0_ref[...], bese0_ref[...])
        h = jax.nn.relu(_dot(h, Wse1_ref[...]) + bse1_ref[...])
        d = jax.nn.relu(_dot(h, Wd0_ref[...]) + bd0_ref[...])
        d = _layernorm(d, gd0_ref[...], bed0_ref[...])
        emb = jax.nn.relu(_dot(d, Wd1_ref[...]) + bd1_ref[...])
        hw1 = _dot(x_ref[...], W1x_ref[...]) + _dot(emb, W1e_ref[...])
        deg = dacc_ref[0, :, :1] + dacc_ref[1, :, :1] + 1.0
        dinv = lax.rsqrt(deg)
        hst_ref[:, :CONV] = hw1 * dinv
        hst_ref[:, CONV:] = hw1 * (dinv * dinv)
        dinv_ref[...] = dinv

    fullspec = lambda shape: pl.BlockSpec(shape, lambda i: (0,) * len(shape))
    return pl.pallas_call(
        body,
        grid=(grid,),
        in_specs=[
            pl.BlockSpec((TB, 2), lambda i: (i, 0)),
            pl.BlockSpec((TB, 6), lambda i: (i, 0)),
            pl.BlockSpec((NC, TB, L), lambda i: (0, i, 0)),
            fullspec((2, 128)), fullspec((1, 128)), fullspec((1, 128)),
            fullspec((1, 128)), fullspec((128, 128)), fullspec((1, 128)),
            fullspec((128, 64)), fullspec((1, 64)), fullspec((1, 64)),
            fullspec((1, 64)), fullspec((64, 16)), fullspec((1, 16)),
            fullspec((6, 64)), fullspec((16, 64)),
        ],
        out_specs=[
            pl.BlockSpec((TB, 128), lambda i: (i, 0)),
            pl.BlockSpec((TB, 1), lambda i: (i, 0)),
        ],
        out_shape=[
            jax.ShapeDtypeStruct((NPAD, 128), jnp.float32),
            jax.ShapeDtypeStruct((NPAD, 1), jnp.float32),
        ],
    )(coords_p, x_p, dacc, W_se0, b_se0, g_se0, be_se0, W_se1, b_se1,
      W_d0, b_d0, g_d0, be_d0, W_d1, b_d1, W1x, W1e)


def _tc_stage2(acc1, hst1, dinv, b1, W2):
    grid = NPAD // TB

    def body(acc_ref, hst_ref, dinv_ref, b1_ref, W2_ref, hst2_ref):
        dinv = dinv_ref[...]
        z1 = jax.nn.relu(acc_ref[:, :CONV] * dinv + hst_ref[:, CONV:]
                         + b1_ref[...])
        hw2 = _dot(z1, W2_ref[...])
        hst2_ref[:, :CONV] = hw2 * dinv
        hst2_ref[:, CONV:] = hw2 * (dinv * dinv)

    fullspec = lambda shape: pl.BlockSpec(shape, lambda i: (0,) * len(shape))
    return pl.pallas_call(
        body,
        grid=(grid,),
        in_specs=[
            pl.BlockSpec((TB, 128), lambda i: (i, 0)),
            pl.BlockSpec((TB, 128), lambda i: (i, 0)),
            pl.BlockSpec((TB, 1), lambda i: (i, 0)),
            fullspec((1, 64)), fullspec((64, 64)),
        ],
        out_specs=pl.BlockSpec((TB, 128), lambda i: (i, 0)),
        out_shape=jax.ShapeDtypeStruct((NPAD, 128), jnp.float32),
    )(acc1, hst1, dinv, b1, W2)


def _tc_stage3(acc2, hst2, dinv, b2, Wfc, bfc, n):
    grid = NPAD // TB

    def body(acc_ref, hst_ref, dinv_ref, b2_ref, Wfc_ref, bfc_ref, out_ref):
        dinv = dinv_ref[...]
        z2 = jax.nn.relu(acc_ref[:, :CONV] * dinv + hst_ref[:, CONV:]
                         + b2_ref[...])
        out_ref[...] = _dot(z2, Wfc_ref[...]) + bfc_ref[...]

    fullspec = lambda shape: pl.BlockSpec(shape, lambda i: (0,) * len(shape))
    return pl.pallas_call(
        body,
        grid=(grid,),
        in_specs=[
            pl.BlockSpec((TB, 128), lambda i: (i, 0)),
            pl.BlockSpec((TB, 128), lambda i: (i, 0)),
            pl.BlockSpec((TB, 1), lambda i: (i, 0)),
            fullspec((1, 64)), fullspec((64, 1)), fullspec((1, 1)),
        ],
        out_specs=pl.BlockSpec((TB, 1), lambda i: (i, 0)),
        out_shape=jax.ShapeDtypeStruct((n, 1), jnp.float32),
    )(acc2, hst2, dinv, b2, Wfc, bfc)


def kernel(x, coords, edge_index, W_se0, b_se0, g_se0, be_se0, W_se1, b_se1,
           W_d0, b_d0, g_d0, be_d0, W_d1, b_d1, W1, b1, W2, b2, Wfc, bfc):
    n = x.shape[0]
    e = edge_index.shape[1]
    assert n < NPAD

    coords_p = jnp.zeros((NPAD, 2), jnp.float32).at[:n].set(coords)
    x_p = jnp.zeros((NPAD, 6), jnp.float32).at[:n].set(x)

    epad = ((e + 2 * NS * EPG - 1) // (2 * NS * EPG)) * (2 * NS * EPG)
    src = jnp.pad(edge_index[0], (0, epad - e),
                  constant_values=n).reshape(epad // EPG, EPG)
    dst = jnp.pad(edge_index[1], (0, epad - e),
                  constant_values=n).reshape(epad // EPG, EPG)

    row = lambda v: v.reshape(1, -1)

    dacc = _sc_degree(dst)
    hst1, dinv = _tc_stage1(
        coords_p, x_p, dacc,
        W_se0, row(b_se0), row(g_se0), row(be_se0),
        W_se1, row(b_se1), W_d0, row(b_d0), row(g_d0), row(be_d0),
        W_d1, row(b_d1), W1[:6], W1[6:])

    acc1 = _sc_edge_agg(hst1.reshape(NPAD * 8, L), src, dst)
    hst2 = _tc_stage2(acc1, hst1, dinv, row(b1), W2)

    acc2 = _sc_edge_agg(hst2.reshape(NPAD * 8, L), src, dst)
    return _tc_stage3(acc2, hst2, dinv, row(b2), Wfc, row(bfc), n)


# speedup vs baseline: 1.3360x; 1.0752x over previous
import functools

import jax
import jax.numpy as jnp
from jax import lax
from jax.experimental import pallas as pl
from jax.experimental.pallas import tpu as pltpu
from jax.experimental.pallas import tpu_sc as plsc

CONV = 64
NC = 2
NS = 16
L = 16
EPG = 256
SBG = 8
NPAD = 49 * 2048
TB = 2048

_SC_PARAMS = None


def _sc_mesh():
    return plsc.VectorSubcoreMesh(core_axis_name="c", subcore_axis_name="s",
                                  num_cores=NC, num_subcores=NS)


def _sc_compiler_params():
    return pltpu.CompilerParams(use_tc_tiling_on_sc=False)


def _zero_acc_slice(acc, buf, s, SL):
    @pl.loop(0, EPG)
    def _zero(i):
        buf[i, :] = jnp.zeros((L,), jnp.float32)

    full, rem = SL // EPG, SL % EPG
    for k in range(full):
        pltpu.sync_copy(buf, acc.at[pl.ds(s * SL + k * EPG, EPG)])
    if rem:
        pltpu.sync_copy(buf.at[pl.ds(0, rem)],
                        acc.at[pl.ds(s * SL + full * EPG, rem)])


def _sc_degree(dst2):
    RTOT = dst2.shape[0]
    R_sc = RTOT // NC
    RT = R_sc // NS
    SL = NPAD // NS

    @functools.partial(
        pl.kernel,
        out_type=jax.ShapeDtypeStruct((NC, NPAD, L), jnp.float32),
        mesh=_sc_mesh(),
        scratch_types=[
            pltpu.VMEM_SHARED((NPAD, L), jnp.float32),
            pltpu.VMEM((EPG, L), jnp.float32),
            pltpu.VMEM((2, EPG), jnp.int32),
            pltpu.SemaphoreType.DMA,
            pltpu.SemaphoreType.DMA,
        ],
        compiler_params=_sc_compiler_params(),
    )
    def deg_kernel(dst_hbm, out_hbm, acc, ones, dstv, semA, semB):
        c = lax.axis_index("c")
        s = lax.axis_index("s")

        _zero_acc_slice(acc, ones, s, SL)

        @pl.loop(0, EPG)
        def _fill(i):
            ones[i, :] = jnp.full((L,), 1.0, jnp.float32)

        plsc.subcore_barrier()

        base = c * R_sc + s * RT
        sem = (semA, semB)

        @pl.loop(0, RT // 2)
        def _edges(m):
            r0 = base + m * 2
            pltpu.sync_copy(dst_hbm.at[pl.ds(r0, 2)], dstv)
            for b in range(2):
                pltpu.async_copy(ones, acc.at[dstv.at[b]], sem[b], add=True)
            for b in range(2):
                pltpu.make_async_copy(ones, acc.at[dstv.at[b]],
                                      sem[b]).wait()

        plsc.subcore_barrier()
        pltpu.sync_copy(acc.at[pl.ds(s * SL, SL)],
                        out_hbm.at[c].at[pl.ds(s * SL, SL)])

    return deg_kernel(dst2)


def _sc_edge_agg(table, src2, dst2):
    RTOT = src2.shape[0]
    RT = RTOT // NS
    SB = RT // SBG
    SL = NPAD // NS
    TSPAN = NPAD * 8 - 7

    @functools.partial(
        pl.kernel,
        out_type=jax.ShapeDtypeStruct((NPAD, 128), jnp.float32),
        mesh=_sc_mesh(),
        scratch_types=[
            pltpu.VMEM_SHARED((NPAD, L), jnp.float32),
            pltpu.VMEM((EPG, L), jnp.float32),
            pltpu.VMEM((EPG, L), jnp.float32),
            pltpu.VMEM((EPG, L), jnp.float32),
            pltpu.VMEM((EPG, L), jnp.float32),
            pltpu.VMEM((SBG, EPG), jnp.int32),
            pltpu.VMEM((SBG, EPG), jnp.int32),
            pltpu.VMEM((SBG, EPG), jnp.int32),
            pltpu.VMEM((SBG, EPG), jnp.int32),
            pltpu.SemaphoreType.DMA,
            pltpu.SemaphoreType.DMA,
            pltpu.SemaphoreType.DMA,
            pltpu.SemaphoreType.DMA,
            pltpu.SemaphoreType.DMA,
            pltpu.SemaphoreType.DMA,
            pltpu.SemaphoreType.DMA,
            pltpu.SemaphoreType.DMA,
        ],
        compiler_params=_sc_compiler_params(),
    )
    def agg_kernel(table_hbm, src_hbm, dst_hbm, out_hbm,
                   acc, rows0, rows1, rows2, rows3,
                   srcv0, srcv1, dstv0, dstv1,
                   gsem0, gsem1, gsem2, gsem3,
                   ssem0, ssem1, ssem2, ssem3):
        c = lax.axis_index("c")
        s = lax.axis_index("s")
        rows = (rows0, rows1, rows2, rows3)
        srcv = (srcv0, srcv1)
        dstv = (dstv0, dstv1)
        gsem = (gsem0, gsem1, gsem2, gsem3)
        ssem = (ssem0, ssem1, ssem2, ssem3)
        base = s * RT

        for p in range(2):
            q = c * 2 + p

            _zero_acc_slice(acc, rows0, s, SL)
            plsc.subcore_barrier()

            tab = table_hbm.at[pl.ds(q, TSPAN)]

            def fetch(m, ip):
                r0 = base + m * SBG
                pltpu.sync_copy(src_hbm.at[pl.ds(r0, SBG)], srcv[ip])
                pltpu.sync_copy(dst_hbm.at[pl.ds(r0, SBG)], dstv[ip])
                for j in range(SBG):
                    @pl.loop(0, EPG // L, unroll=8)
                    def _adj(i):
                        srcv[ip][j, pl.ds(i * L, L)] = (
                            srcv[ip][j, pl.ds(i * L, L)] * 8)

            def g_fire(b, ip, j):
                pltpu.async_copy(tab.at[srcv[ip].at[j]], rows[b], gsem[b])

            def g_wait(b):
                pltpu.make_async_copy(tab.at[srcv[0].at[0]], rows[b],
                                      gsem[b]).wait()

            def s_fire(b, ip, j):
                pltpu.async_copy(rows[b], acc.at[dstv[ip].at[j]],
                                 ssem[b], add=True)

            def s_wait(b):
                pltpu.make_async_copy(rows[b], acc.at[dstv[0].at[0]],
                                      ssem[b]).wait()

            def sb_body(m, ip):
                fetch(m, ip)
                s_wait(0); g_fire(0, ip, 0)
                g_wait(2); s_fire(2, 1 - ip, 6)
                s_wait(1); g_fire(1, ip, 1)
                g_wait(3); s_fire(3, 1 - ip, 7)
                s_wait(2); g_fire(2, ip, 2)
                g_wait(0); s_fire(0, ip, 0)
                s_wait(3); g_fire(3, ip, 3)
                g_wait(1); s_fire(1, ip, 1)
                s_wait(0); g_fire(0, ip, 4)
                g_wait(2); s_fire(2, ip, 2)
                s_wait(1); g_fire(1, ip, 5)
                g_wait(3); s_fire(3, ip, 3)
                s_wait(2); g_fire(2, ip, 6)
                g_wait(0); s_fire(0, ip, 4)
                s_wait(3); g_fire(3, ip, 7)
                g_wait(1); s_fire(1, ip, 5)

            fetch(0, 0)
            g_fire(0, 0, 0)
            g_fire(1, 0, 1)
            g_fire(2, 0, 2)
            g_wait(0); s_fire(0, 0, 0)
            g_fire(3, 0, 3)
            g_wait(1); s_fire(1, 0, 1)
            s_wait(0); g_fire(0, 0, 4)
            g_wait(2); s_fire(2, 0, 2)
            s_wait(1); g_fire(1, 0, 5)
            g_wait(3); s_fire(3, 0, 3)
            s_wait(2); g_fire(2, 0, 6)
            g_wait(0); s_fire(0, 0, 4)
            s_wait(3); g_fire(3, 0, 7)
            g_wait(1); s_fire(1, 0, 5)

            @pl.loop(0, (SB - 1) // 2)
            def _edges(t):
                sb_body(2 * t + 1, 1)
                sb_body(2 * t + 2, 0)

            if (SB - 1) % 2:
                sb_body(SB - 1, (SB - 1) % 2)

            ipl = (SB - 1) % 2
            g_wait(2); s_fire(2, ipl, 6)
            g_wait(3); s_fire(3, ipl, 7)
            for b in range(4):
                s_wait(b)

            plsc.subcore_barrier()
            pltpu.sync_copy(acc.at[pl.ds(s * SL, SL)],
                            out_hbm.at[pl.ds(s * SL, SL), pl.ds(q * L, L)])
            plsc.subcore_barrier()

    return agg_kernel(table, src2, dst2)


def _layernorm(h, g, b):
    m = jnp.mean(h, axis=-1, keepdims=True)
    v = jnp.mean((h - m) * (h - m), axis=-1, keepdims=True)
    return (h - m) * lax.rsqrt(v + 1e-5) * g + b


def _dot(a, b):
    return jnp.dot(a, b, preferred_element_type=jnp.float32)


def _tc_stage1(coords_p, x_p, dacc, W_se0, b_se0, g_se0, be_se0,
               W_se1, b_se1, W_d0, b_d0, g_d0, be_d0, W_d1, b_d1, W1x, W1e):
    grid = NPAD // TB

    def body(coords_ref, x_ref, dacc_ref, Wse0_ref, bse0_ref, gse0_ref,
             bese0_ref, Wse1_ref, bse1_ref, Wd0_ref, bd0_ref, gd0_ref,
             bed0_ref, Wd1_ref, bd1_ref, W1x_ref, W1e_ref,
             hst_ref, dinv_ref):
        h = jax.nn.relu(_dot(coords_ref[...], Wse0_ref[...]) + bse0_ref[...])
        h = _layernorm(h, gse0_ref[...], bese0_ref[...])
        h = jax.nn.relu(_dot(h, Wse1_ref[...]) + bse1_ref[...])
        d = jax.nn.relu(_dot(h, Wd0_ref[...]) + bd0_ref[...])
        d = _layernorm(d, gd0_ref[...], bed0_ref[...])
        emb = jax.nn.relu(_dot(d, Wd1_ref[...]) + bd1_ref[...])
        hw1 = _dot(x_ref[...], W1x_ref[...]) + _dot(emb, W1e_ref[...])
        deg = dacc_ref[0, :, :1] + dacc_ref[1, :, :1] + 1.0
        dinv = lax.rsqrt(deg)
        hst_ref[:, :CONV] = hw1 * dinv
        hst_ref[:, CONV:] = hw1 * (dinv * dinv)
        dinv_ref[...] = dinv

    fullspec = lambda shape: pl.BlockSpec(shape, lambda i: (0,) * len(shape))
    return pl.pallas_call(
        body,
        grid=(grid,),
        in_specs=[
            pl.BlockSpec((TB, 2), lambda i: (i, 0)),
            pl.BlockSpec((TB, 6), lambda i: (i, 0)),
            pl.BlockSpec((NC, TB, L), lambda i: (0, i, 0)),
            fullspec((2, 128)), fullspec((1, 128)), fullspec((1, 128)),
            fullspec((1, 128)), fullspec((128, 128)), fullspec((1, 128)),
            fullspec((128, 64)), fullspec((1, 64)), fullspec((1, 64)),
            fullspec((1, 64)), fullspec((64, 16)), fullspec((1, 16)),
            fullspec((6, 64)), fullspec((16, 64)),
        ],
        out_specs=[
            pl.BlockSpec((TB, 128), lambda i: (i, 0)),
            pl.BlockSpec((TB, 1), lambda i: (i, 0)),
        ],
        out_shape=[
            jax.ShapeDtypeStruct((NPAD, 128), jnp.float32),
            jax.ShapeDtypeStruct((NPAD, 1), jnp.float32),
        ],
    )(coords_p, x_p, dacc, W_se0, b_se0, g_se0, be_se0, W_se1, b_se1,
      W_d0, b_d0, g_d0, be_d0, W_d1, b_d1, W1x, W1e)


def _tc_stage2(acc1, hst1, dinv, b1, W2):
    grid = NPAD // TB

    def body(acc_ref, hst_ref, dinv_ref, b1_ref, W2_ref, hst2_ref):
        dinv = dinv_ref[...]
        z1 = jax.nn.relu(acc_ref[:, :CONV] * dinv + hst_ref[:, CONV:]
                         + b1_ref[...])
        hw2 = _dot(z1, W2_ref[...])
        hst2_ref[:, :CONV] = hw2 * dinv
        hst2_ref[:, CONV:] = hw2 * (dinv * dinv)

    fullspec = lambda shape: pl.BlockSpec(shape, lambda i: (0,) * len(shape))
    return pl.pallas_call(
        body,
        grid=(grid,),
        in_specs=[
            pl.BlockSpec((TB, 128), lambda i: (i, 0)),
            pl.BlockSpec((TB, 128), lambda i: (i, 0)),
            pl.BlockSpec((TB, 1), lambda i: (i, 0)),
            fullspec((1, 64)), fullspec((64, 64)),
        ],
        out_specs=pl.BlockSpec((TB, 128), lambda i: (i, 0)),
        out_shape=jax.ShapeDtypeStruct((NPAD, 128), jnp.float32),
    )(acc1, hst1, dinv, b1, W2)


def _tc_stage3(acc2, hst2, dinv, b2, Wfc, bfc, n):
    grid = NPAD // TB

    def body(acc_ref, hst_ref, dinv_ref, b2_ref, Wfc_ref, bfc_ref, out_ref):
        dinv = dinv_ref[...]
        z2 = jax.nn.relu(acc_ref[:, :CONV] * dinv + hst_ref[:, CONV:]
                         + b2_ref[...])
        out_ref[...] = _dot(z2, Wfc_ref[...]) + bfc_ref[...]

    fullspec = lambda shape: pl.BlockSpec(shape, lambda i: (0,) * len(shape))
    return pl.pallas_call(
        body,
        grid=(grid,),
        in_specs=[
            pl.BlockSpec((TB, 128), lambda i: (i, 0)),
            pl.BlockSpec((TB, 128), lambda i: (i, 0)),
            pl.BlockSpec((TB, 1), lambda i: (i, 0)),
            fullspec((1, 64)), fullspec((64, 1)), fullspec((1, 1)),
        ],
        out_specs=pl.BlockSpec((TB, 1), lambda i: (i, 0)),
        out_shape=jax.ShapeDtypeStruct((n, 1), jnp.float32),
    )(acc2, hst2, dinv, b2, Wfc, bfc)


def kernel(x, coords, edge_index, W_se0, b_se0, g_se0, be_se0, W_se1, b_se1,
           W_d0, b_d0, g_d0, be_d0, W_d1, b_d1, W1, b1, W2, b2, Wfc, bfc):
    n = x.shape[0]
    e = edge_index.shape[1]
    assert n < NPAD

    coords_p = jnp.zeros((NPAD, 2), jnp.float32).at[:n].set(coords)
    x_p = jnp.zeros((NPAD, 6), jnp.float32).at[:n].set(x)

    epad = ((e + 2 * NS * EPG - 1) // (2 * NS * EPG)) * (2 * NS * EPG)
    src = jnp.pad(edge_index[0], (0, epad - e),
                  constant_values=n).reshape(epad // EPG, EPG)
    dst = jnp.pad(edge_index[1], (0, epad - e),
                  constant_values=n).reshape(epad // EPG, EPG)

    row = lambda v: v.reshape(1, -1)

    dacc = _sc_degree(dst)
    hst1, dinv = _tc_stage1(
        coords_p, x_p, dacc,
        W_se0, row(b_se0), row(g_se0), row(be_se0),
        W_se1, row(b_se1), W_d0, row(b_d0), row(g_d0), row(be_d0),
        W_d1, row(b_d1), W1[:6], W1[6:])

    acc1 = _sc_edge_agg(hst1.reshape(NPAD * 8, L), src, dst)
    hst2 = _tc_stage2(acc1, hst1, dinv, row(b1), W2)

    acc2 = _sc_edge_agg(hst2.reshape(NPAD * 8, L), src, dst)
    return _tc_stage3(acc2, hst2, dinv, row(b2), Wfc, row(bfc), n)
